# Initial kernel scaffold; baseline (speedup 1.0000x reference)
import jax
import jax.numpy as jnp
from jax.experimental import pallas as pl


def kernel(batch_inputs, emb, W1, b1, W2, b2):
    raise NotImplementedError("write your pallas kernel here")


# speedup vs baseline: 11.6480x; 11.6480x over previous
import functools

import jax
import jax.numpy as jnp
from jax import lax
from jax.experimental import pallas as pl
from jax.experimental.pallas import tpu as pltpu
from jax.experimental.pallas import tpu_sc as plsc

S = 3
L = 50
B = 16384
D = 64
R = S * B
NC, NS = 2, 16
NW = NC * NS
TILE_ROWS = R // NW
ROWS_PER_STEP = 2
G = ROWS_PER_STEP * L
STEPS = TILE_ROWS // ROWS_PER_STEP
OUT_BLK_STEPS = 128
OUT_BLK = OUT_BLK_STEPS * ROWS_PER_STEP
NBLK = STEPS // OUT_BLK_STEPS

_sc_mesh = plsc.VectorSubcoreMesh(
    core_axis_name="c", subcore_axis_name="s", num_cores=NC, num_subcores=NS
)


@functools.partial(
    pl.kernel,
    out_type=jax.ShapeDtypeStruct((R, D), jnp.float32),
    mesh=_sc_mesh,
    scratch_types=[
        pltpu.VMEM((OUT_BLK_STEPS, G), jnp.int32),
        pltpu.VMEM((G, D), jnp.float32),
        pltpu.VMEM((OUT_BLK, D), jnp.float32),
        pltpu.SemaphoreType.DMA,
    ],
    compiler_params=pltpu.CompilerParams(use_tc_tiling_on_sc=False),
)
def _sc_pool(idx_hbm, emb_hbm, out_hbm, idxbuf, rows, outbuf, sem):
    wid = lax.axis_index("s") * NC + lax.axis_index("c")
    for blk in range(NBLK):
        pltpu.sync_copy(idx_hbm.at[wid, pl.ds(blk * OUT_BLK_STEPS, OUT_BLK_STEPS)], idxbuf)

        def step(j, carry):
            pltpu.async_copy(emb_hbm.at[idxbuf.at[j]], rows, sem).wait()

            def lbody(li, acc):
                accs = list(acc)
                for k in range(5):
                    l = li * 5 + k
                    for c in range(ROWS_PER_STEP):
                        for g in range(D // 16):
                            accs[c * 4 + g] = accs[c * 4 + g] + rows[
                                L * c + l, pl.ds(g * 16, 16)
                            ]
                return tuple(accs)

            zero = jnp.zeros((16,), jnp.float32)
            acc = lax.fori_loop(0, L // 5, lbody, (zero,) * (ROWS_PER_STEP * 4))
            for c in range(ROWS_PER_STEP):
                for g in range(D // 16):
                    outbuf[j * ROWS_PER_STEP + c, pl.ds(g * 16, 16)] = acc[c * 4 + g]
            return carry

        lax.fori_loop(0, OUT_BLK_STEPS, step, 0)
        pltpu.sync_copy(
            outbuf, out_hbm.at[pl.ds(wid * TILE_ROWS + blk * OUT_BLK, OUT_BLK)]
        )


def _mlp_body(pool_ref, w1_ref, b1_ref, w2_ref, b2_ref, out_ref):
    f32 = jnp.float32
    h = (
        jnp.dot(pool_ref[0], w1_ref[0:D, :], preferred_element_type=f32)
        + jnp.dot(pool_ref[1], w1_ref[D : 2 * D, :], preferred_element_type=f32)
        + jnp.dot(pool_ref[2], w1_ref[2 * D : 3 * D, :], preferred_element_type=f32)
    )
    h = h * (1.0 / B) + b1_ref[0]
    h = jnp.maximum(h, 0.0)
    o = jnp.dot(h, w2_ref[...], preferred_element_type=f32) + b2_ref[0]
    m = jnp.max(o, axis=1, keepdims=True)
    e = o - m
    out_ref[...] = e - jnp.log(jnp.sum(jnp.exp(e), axis=1, keepdims=True))


def _tc_mlp(pooled3, W1, b1, W2, b2):
    BLK = 512
    grid = (B // BLK,)
    return pl.pallas_call(
        _mlp_body,
        grid=grid,
        in_specs=[
            pl.BlockSpec((S, BLK, D), lambda i: (0, i, 0)),
            pl.BlockSpec((S * D, 128), lambda i: (0, 0)),
            pl.BlockSpec((1, 128), lambda i: (0, 0)),
            pl.BlockSpec((128, 10), lambda i: (0, 0)),
            pl.BlockSpec((1, 10), lambda i: (0, 0)),
        ],
        out_specs=pl.BlockSpec((BLK, 10), lambda i: (i, 0)),
        out_shape=jax.ShapeDtypeStruct((B, 10), jnp.float32),
    )(pooled3, W1, b1, W2, b2)


def kernel(batch_inputs, emb, W1, b1, W2, b2):
    idx = jnp.transpose(batch_inputs, (0, 2, 1)).reshape(NW, STEPS, G)
    pooled = _sc_pool(idx, emb)
    pooled3 = pooled.reshape(S, B, D)
    return _tc_mlp(pooled3, W1, b1.reshape(1, -1), W2, b2.reshape(1, -1))


# speedup vs baseline: 18.2308x; 1.5651x over previous
import functools

import jax
import jax.numpy as jnp
from jax import lax
from jax.experimental import pallas as pl
from jax.experimental.pallas import tpu as pltpu
from jax.experimental.pallas import tpu_sc as plsc

S = 3
L = 50
B = 16384
D = 64
R = S * B
NC, NS = 2, 16
NW = NC * NS
TILE_ROWS = R // NW
ROWS_PER_STEP = 2
G = ROWS_PER_STEP * L
STEPS = TILE_ROWS // ROWS_PER_STEP
OUT_BLK_STEPS = 128
OUT_BLK = OUT_BLK_STEPS * ROWS_PER_STEP
NBLK = STEPS // OUT_BLK_STEPS

_sc_mesh = plsc.VectorSubcoreMesh(
    core_axis_name="c", subcore_axis_name="s", num_cores=NC, num_subcores=NS
)


@functools.partial(
    pl.kernel,
    out_type=jax.ShapeDtypeStruct((R, D), jnp.float32),
    mesh=_sc_mesh,
    scratch_types=[
        pltpu.VMEM((STEPS, G), jnp.int32),
        pltpu.VMEM((G, D), jnp.float32),
        pltpu.VMEM((G, D), jnp.float32),
        pltpu.VMEM((OUT_BLK, D), jnp.float32),
        pltpu.SemaphoreType.DMA,
        pltpu.SemaphoreType.DMA,
    ],
    compiler_params=pltpu.CompilerParams(use_tc_tiling_on_sc=False),
)
def _sc_pool(idx_hbm, emb_hbm, out_hbm, idxbuf, rows0, rows1, outbuf, sem0, sem1):
    wid = lax.axis_index("s") * NC + lax.axis_index("c")
    pltpu.sync_copy(idx_hbm.at[wid], idxbuf)
    pltpu.async_copy(emb_hbm.at[idxbuf.at[0]], rows0, sem0)
    pltpu.async_copy(emb_hbm.at[idxbuf.at[1]], rows1, sem1)

    def accum(rows, j):
        for c in range(ROWS_PER_STEP):
            accs = [rows[L * c, pl.ds(g * 16, 16)] for g in range(D // 16)]
            for l in range(1, L):
                for g in range(D // 16):
                    accs[g] = accs[g] + rows[L * c + l, pl.ds(g * 16, 16)]
            row = (j % OUT_BLK_STEPS) * ROWS_PER_STEP + c
            for g in range(D // 16):
                outbuf[row, pl.ds(g * 16, 16)] = accs[g]

    def pair(jp, carry):
        j0 = jp * 2
        for par in range(2):
            rows = rows0 if par == 0 else rows1
            sem = sem0 if par == 0 else sem1
            j = j0 + par
            pltpu.make_async_copy(emb_hbm.at[pl.ds(0, G)], rows, sem).wait()
            accum(rows, j)

            @pl.when(j % OUT_BLK_STEPS == OUT_BLK_STEPS - 1)
            def _flush():
                pltpu.sync_copy(
                    outbuf,
                    out_hbm.at[
                        pl.ds(
                            wid * TILE_ROWS + (j // OUT_BLK_STEPS) * OUT_BLK, OUT_BLK
                        )
                    ],
                )

            @pl.when(j + 2 < STEPS)
            def _next():
                pltpu.async_copy(emb_hbm.at[idxbuf.at[j + 2]], rows, sem)

        return carry

    lax.fori_loop(0, STEPS // 2, pair, 0)


def _mlp_body(pool_ref, w1_ref, b1_ref, w2_ref, b2_ref, out_ref):
    f32 = jnp.float32
    h = (
        jnp.dot(pool_ref[0], w1_ref[0:D, :], preferred_element_type=f32)
        + jnp.dot(pool_ref[1], w1_ref[D : 2 * D, :], preferred_element_type=f32)
        + jnp.dot(pool_ref[2], w1_ref[2 * D : 3 * D, :], preferred_element_type=f32)
    )
    h = h * (1.0 / B) + b1_ref[0]
    h = jnp.maximum(h, 0.0)
    o = jnp.dot(h, w2_ref[...], preferred_element_type=f32) + b2_ref[0]
    m = jnp.max(o, axis=1, keepdims=True)
    e = o - m
    out_ref[...] = e - jnp.log(jnp.sum(jnp.exp(e), axis=1, keepdims=True))


def _tc_mlp(pooled3, W1, b1, W2, b2):
    BLK = 512
    grid = (B // BLK,)
    return pl.pallas_call(
        _mlp_body,
        grid=grid,
        in_specs=[
            pl.BlockSpec((S, BLK, D), lambda i: (0, i, 0)),
            pl.BlockSpec((S * D, 128), lambda i: (0, 0)),
            pl.BlockSpec((1, 128), lambda i: (0, 0)),
            pl.BlockSpec((128, 10), lambda i: (0, 0)),
            pl.BlockSpec((1, 10), lambda i: (0, 0)),
        ],
        out_specs=pl.BlockSpec((BLK, 10), lambda i: (i, 0)),
        out_shape=jax.ShapeDtypeStruct((B, 10), jnp.float32),
    )(pooled3, W1, b1, W2, b2)


def kernel(batch_inputs, emb, W1, b1, W2, b2):
    idx = jnp.transpose(batch_inputs, (0, 2, 1)).reshape(NW, STEPS, G)
    pooled = _sc_pool(idx, emb)
    pooled3 = pooled.reshape(S, B, D)
    return _tc_mlp(pooled3, W1, b1.reshape(1, -1), W2, b2.reshape(1, -1))
